# baseline (speedup 1.0000x reference)
import functools

import jax
import jax.numpy as jnp
from jax import lax
from jax.experimental import pallas as pl
from jax.experimental.pallas import tpu as pltpu
from jax.experimental.pallas import tpu_sc as plsc

N_NODES = 100000
IN_C = 128
OUT_C = 128
DENSE_OUT = 256
EMB_DIM = 64

SC_ROWS = 12800
TC_ROWS = N_NODES - SC_ROWS
SLICE_R = 4360
NSLICE = TC_ROWS // SLICE_R
NBUF = 4

NC = 2
NS = 16
NW = NC * NS
ROWS_PER_W = SC_ROWS // NW
NGRP = IN_C // 16
W_ELEMS = ROWS_PER_W * IN_C
SC_START_ELEM = TC_ROWS * IN_C


def _sc_partial_body(x_hbm, out_hbm, buf, accv, sem):
    wid = lax.axis_index("s") * NC + lax.axis_index("c")
    base = SC_START_ELEM + wid * W_ELEMS
    pltpu.async_copy(x_hbm.at[pl.ds(base, W_ELEMS)], buf, sem).wait()

    def row_body(r, carry):
        return tuple(
            carry[c] + buf[pl.ds(r * IN_C + c * 16, 16)]
            for c in range(NGRP))

    acc = lax.fori_loop(0, ROWS_PER_W,
                        row_body,
                        tuple(jnp.zeros((16,), jnp.float32)
                              for _ in range(NGRP)),
                        unroll=5)
    for c in range(NGRP):
        accv[pl.ds(c * 16, 16)] = acc[c]
    pltpu.sync_copy(accv, out_hbm.at[pl.ds(wid * IN_C, IN_C)])


_sc_partial = functools.partial(
    pl.kernel,
    mesh=plsc.VectorSubcoreMesh(core_axis_name="c", subcore_axis_name="s"),
    out_type=jax.ShapeDtypeStruct((NW * IN_C,), jnp.float32),
    scratch_types=[
        pltpu.VMEM((W_ELEMS,), jnp.float32),
        pltpu.VMEM((IN_C,), jnp.float32),
        pltpu.SemaphoreType.DMA,
    ],
)(_sc_partial_body)


def _tc_colsum_kernel(x_hbm, out_ref, bufs, sems):
    def start(k):
        return pltpu.make_async_copy(
            x_hbm.at[pl.ds(k * SLICE_R, SLICE_R), :], bufs.at[k % NBUF],
            sems.at[k % NBUF])

    for k in range(NBUF):
        start(k).start()

    acc = jnp.zeros((1, IN_C), jnp.float32)
    for k in range(NSLICE):
        start(k).wait()
        acc = acc + jnp.sum(bufs[k % NBUF], axis=0, keepdims=True)
        if k + NBUF < NSLICE:
            start(k + NBUF).start()
    out_ref[...] = acc


def _finish_kernel(p_ref, t_ref, x0_ref, w0_ref, w1_ref, w2_ref, cb_ref,
                   dw_ref, db_ref, ew_ref, eb_ref, y_ref):
    inv_n = 1.0 / N_NODES
    colsum = jnp.sum(p_ref[...], axis=0, keepdims=True) + t_ref[...]
    colmean = colsum * inv_n
    x0 = x0_ref[0:1, :]
    w_mean = w0_ref[...] - w2_ref[...]
    w_corr = 18.0 * w2_ref[...] - 3.0 * w1_ref[...]
    dn = (((1,), (1,)), ((), ()))
    pooled = (
        jax.lax.dot_general(colmean, w_mean, dn,
                            preferred_element_type=jnp.float32)
        + inv_n * jax.lax.dot_general(x0, w_corr, dn,
                                      preferred_element_type=jnp.float32)
        + cb_ref[...]
    )
    h = jax.lax.dot_general(pooled, dw_ref[...], dn,
                            preferred_element_type=jnp.float32) + db_ref[...]
    y = jax.lax.dot_general(h, ew_ref[...], dn,
                            preferred_element_type=jnp.float32) + eb_ref[...]
    y_ref[...] = y


@jax.jit
def kernel(x, cheb_W0, cheb_W1, cheb_W2, cheb_b, dense_W, dense_b, emb_W,
           emb_b):
    sc_partials = _sc_partial(x.reshape(-1)).reshape(NW, IN_C)

    tc_partial = pl.pallas_call(
        _tc_colsum_kernel,
        grid=(1,),
        in_specs=[pl.BlockSpec(memory_space=pl.ANY)],
        out_specs=pl.BlockSpec((1, IN_C), lambda i: (0, 0)),
        out_shape=jax.ShapeDtypeStruct((1, IN_C), jnp.float32),
        scratch_shapes=[
            pltpu.VMEM((NBUF, SLICE_R, IN_C), jnp.float32),
            pltpu.SemaphoreType.DMA((NBUF,)),
        ],
    )(x)

    cb = cheb_b.reshape(1, OUT_C)
    db = dense_b.reshape(1, DENSE_OUT)
    eb = emb_b.reshape(1, EMB_DIM)

    full = lambda shape: pl.BlockSpec(shape, lambda i: (0,) * len(shape))
    return pl.pallas_call(
        _finish_kernel,
        grid=(1,),
        in_specs=[
            full((NW, IN_C)),
            full((1, IN_C)),
            pl.BlockSpec((8, IN_C), lambda i: (0, 0)),
            full((OUT_C, IN_C)),
            full((OUT_C, IN_C)),
            full((OUT_C, IN_C)),
            full((1, OUT_C)),
            full((DENSE_OUT, OUT_C)),
            full((1, DENSE_OUT)),
            full((EMB_DIM, DENSE_OUT)),
            full((1, EMB_DIM)),
        ],
        out_specs=pl.BlockSpec((1, EMB_DIM), lambda i: (0, 0)),
        out_shape=jax.ShapeDtypeStruct((1, EMB_DIM), jnp.float32),
    )(sc_partials, tc_partial, x, cheb_W0, cheb_W1, cheb_W2, cb, dense_W, db,
      emb_W, eb)


# speedup vs baseline: 1.8988x; 1.8988x over previous
import functools

import jax
import jax.numpy as jnp
from jax.experimental import pallas as pl
from jax.experimental.pallas import tpu as pltpu

N_NODES = 100000
IN_C = 128
OUT_C = 128
DENSE_OUT = 256
EMB_DIM = 64

SLICE_R = 4000
NSLICE = N_NODES // SLICE_R
NBUF = 5


def _gnn_kernel(x_hbm, w0_ref, w1_ref, w2_ref, cb_ref, dw_ref, db_ref,
                ew_ref, eb_ref, y_ref, bufs, sems):
    def start(k):
        return pltpu.make_async_copy(
            x_hbm.at[pl.ds(k * SLICE_R, SLICE_R), :], bufs.at[k % NBUF],
            sems.at[k % NBUF])

    for k in range(NBUF):
        start(k).start()

    acc = jnp.zeros((1, IN_C), jnp.float32)
    x0 = None
    for k in range(NSLICE):
        start(k).wait()
        if k == 0:
            x0 = bufs[0, 0:1, :]
        acc = acc + jnp.sum(bufs[k % NBUF], axis=0, keepdims=True)
        if k + NBUF < NSLICE:
            start(k + NBUF).start()

    inv_n = 1.0 / N_NODES
    colmean = acc * inv_n
    w_mean = w0_ref[...] - w2_ref[...]
    w_corr = 18.0 * w2_ref[...] - 3.0 * w1_ref[...]
    dn = (((1,), (1,)), ((), ()))
    pooled = (
        jax.lax.dot_general(colmean, w_mean, dn,
                            preferred_element_type=jnp.float32)
        + inv_n * jax.lax.dot_general(x0, w_corr, dn,
                                      preferred_element_type=jnp.float32)
        + cb_ref[...]
    )
    h = jax.lax.dot_general(pooled, dw_ref[...], dn,
                            preferred_element_type=jnp.float32) + db_ref[...]
    y = jax.lax.dot_general(h, ew_ref[...], dn,
                            preferred_element_type=jnp.float32) + eb_ref[...]
    y_ref[...] = y


@jax.jit
def kernel(x, cheb_W0, cheb_W1, cheb_W2, cheb_b, dense_W, dense_b, emb_W,
           emb_b):
    cb = cheb_b.reshape(1, OUT_C)
    db = dense_b.reshape(1, DENSE_OUT)
    eb = emb_b.reshape(1, EMB_DIM)

    full = lambda shape: pl.BlockSpec(shape, lambda i: (0,) * len(shape))
    return pl.pallas_call(
        _gnn_kernel,
        grid=(1,),
        in_specs=[
            pl.BlockSpec(memory_space=pl.ANY),
            full((OUT_C, IN_C)),
            full((OUT_C, IN_C)),
            full((OUT_C, IN_C)),
            full((1, OUT_C)),
            full((DENSE_OUT, OUT_C)),
            full((1, DENSE_OUT)),
            full((EMB_DIM, DENSE_OUT)),
            full((1, EMB_DIM)),
        ],
        out_specs=pl.BlockSpec((1, EMB_DIM), lambda i: (0, 0)),
        out_shape=jax.ShapeDtypeStruct((1, EMB_DIM), jnp.float32),
        scratch_shapes=[
            pltpu.VMEM((NBUF, SLICE_R, IN_C), jnp.float32),
            pltpu.SemaphoreType.DMA((NBUF,)),
        ],
    )(x, cheb_W0, cheb_W1, cheb_W2, cb, dense_W, db, emb_W, eb)
